# baseline (speedup 1.0000x reference)
import functools

import jax
import jax.numpy as jnp
from jax import lax
from jax.experimental import pallas as pl
from jax.experimental.pallas import tpu as pltpu
from jax.experimental.pallas import tpu_sc as plsc

_N = 10000
_M = 16
_D = 64
_NBR = 41
_H = 128
_NW = 32
_CHUNK = 128
_AB = 400
_NB = _N // _AB


_G = 3


def _sc_gather(table, idx_flat):
    e = idx_flat.shape[0]
    v, dpad = table.shape
    assert e % _CHUNK == 0 and dpad == 2 * _D
    c_total = e // _CHUNK
    idx2 = idx_flat.reshape(c_total, _CHUNK)
    stride = -(-c_total // _NW)
    n_per = -(-stride // (2 * _G)) * (2 * _G)
    assert n_per <= c_total
    n_groups = n_per // _G

    mesh = plsc.VectorSubcoreMesh(core_axis_name="c", subcore_axis_name="s")

    @functools.partial(
        pl.kernel,
        mesh=mesh,
        out_type=jax.ShapeDtypeStruct((e, 2 * _D), jnp.float32),
        scratch_types=[
            pltpu.VMEM((_G, _CHUNK), jnp.int32),
            pltpu.VMEM((_G, _CHUNK), jnp.int32),
            pltpu.VMEM((_G * _CHUNK, 2 * _D), jnp.float32),
            pltpu.VMEM((_G * _CHUNK, 2 * _D), jnp.float32),
            pltpu.SemaphoreType.DMA,
            pltpu.SemaphoreType.DMA,
            pltpu.SemaphoreType.DMA,
        ],
    )
    def k(table_hbm, idx_hbm, out_hbm, idx0, idx1, rows0, rows1, sem_g,
          sem_o0, sem_o1):
        wid = lax.axis_index("s") * 2 + lax.axis_index("c")
        start = jnp.minimum(wid * stride, c_total - n_per)
        idx_b = (idx0, idx1)
        rows_b = (rows0, rows1)
        sem_b = (sem_o0, sem_o1)

        def run_group(g, b, first):
            if not first:
                pltpu.make_async_copy(
                    out_hbm.at[pl.ds(0, _G * _CHUNK)], rows_b[b], sem_b[b]
                ).wait()
            cs = start + g * _G
            for kk in range(_G):
                pltpu.sync_copy(idx_hbm.at[cs + kk], idx_b[b].at[kk])
            ds = []
            for kk in range(_G):
                ds.append(pltpu.async_copy(
                    table_hbm.at[idx_b[b].at[kk]],
                    rows_b[b].at[pl.ds(kk * _CHUNK, _CHUNK)],
                    sem_g,
                ))
            for d in ds:
                d.wait()
            pltpu.async_copy(
                rows_b[b], out_hbm.at[pl.ds(cs * _CHUNK, _G * _CHUNK)],
                sem_b[b],
            )

        run_group(0, 0, True)
        run_group(1, 1, True)

        def body(i2, carry):
            run_group(i2 * 2, 0, False)
            run_group(i2 * 2 + 1, 1, False)
            return carry

        lax.fori_loop(1, n_groups // 2, body, 0)
        for b in range(2):
            pltpu.make_async_copy(
                out_hbm.at[pl.ds(0, _G * _CHUNK)], rows_b[b], sem_b[b]
            ).wait()

    return k(table, idx2)


def _emb_body(af_ref, w_ref, b_ref, out_ref):
    x = (
        jnp.dot(af_ref[:], w_ref[:], preferred_element_type=jnp.float32)
        + b_ref[:]
    )
    out_ref[:] = jnp.concatenate(
        [x, jnp.zeros((x.shape[0], _D), jnp.float32)], axis=1
    )


def _embed(atom_fea, w, b):
    return pl.pallas_call(
        _emb_body,
        out_shape=jax.ShapeDtypeStruct((_N, 2 * _D), jnp.float32),
    )(atom_fea, w, b.reshape(1, _D))


def _conv_body(x_ref, g_ref, nbr_ref, ws_ref, wn_ref, we_ref, b_ref,
               g1_ref, be1_ref, g2_ref, be2_ref, out_ref, s_scr, st1, st2):
    p = pl.program_id(0)
    j = pl.program_id(1)
    row0 = pl.multiple_of(j * _AB, 8)

    def edge_parts():
        xs = x_ref[pl.ds(row0, _AB), pl.ds(0, _D)]
        zself = (
            jnp.dot(xs, ws_ref[:], preferred_element_type=jnp.float32)
            + b_ref[:]
        )
        ze = jnp.dot(g_ref[:], wn_ref[:], preferred_element_type=jnp.float32)
        nbr2 = nbr_ref[:].reshape(_AB * _M, _NBR)
        ze = ze + jnp.dot(
            nbr2, we_ref[:], preferred_element_type=jnp.float32
        )
        return zself, ze

    @pl.when(p == 0)
    def _phase0():
        zself, ze = edge_parts()
        sze = jnp.sum(ze.reshape(_AB, _M, 2 * _D), axis=1)
        s0 = (
            jnp.sum(ze, axis=0, keepdims=True)
            + _M * jnp.sum(zself, axis=0, keepdims=True)
        )

        @pl.when(j == 0)
        def _():
            st1[0:1, :] = s0
            st1[2:3, :] = s0 / float(_AB * _M)

        @pl.when(j > 0)
        def _():
            st1[0:1, :] = st1[0:1, :] + s0

        zc = zself - st1[2:3, :]
        s1 = (
            jnp.sum(ze * ze, axis=0, keepdims=True)
            + 2.0 * jnp.sum(zc * sze, axis=0, keepdims=True)
            + _M * jnp.sum(zc * zc, axis=0, keepdims=True)
        )

        @pl.when(j == 0)
        def _():
            st1[1:2, :] = s1

        @pl.when(j > 0)
        def _():
            st1[1:2, :] = st1[1:2, :] + s1

    @pl.when(p == 1)
    def _phase1():
        cnt = float(_N * _M)
        mu = st1[0:1, :] / cnt
        dm = mu - st1[2:3, :]
        var = st1[1:2, :] / cnt - dm * dm
        inv = lax.rsqrt(var + 1e-5)
        scale = g1_ref[:] * inv

        zself, ze = edge_parts()
        zc3 = ze.reshape(_AB, _M, 2 * _D) + (zself - mu)[:, None, :]
        zn = zc3 * scale[None] + be1_ref[:][None]
        filt = jax.nn.sigmoid(zn[:, :, :_D])
        core = jax.nn.softplus(zn[:, :, _D:])
        sblk = jnp.sum(filt * core, axis=1)
        s_scr[pl.ds(row0, _AB), :] = sblk
        t0 = jnp.sum(sblk, axis=0, keepdims=True)

        @pl.when(j == 0)
        def _():
            st2[0:1, :] = t0
            st2[2:3, :] = t0 / float(_AB)

        @pl.when(j > 0)
        def _():
            st2[0:1, :] = st2[0:1, :] + t0

        sc = sblk - st2[2:3, :]
        t1 = jnp.sum(sc * sc, axis=0, keepdims=True)

        @pl.when(j == 0)
        def _():
            st2[1:2, :] = t1

        @pl.when(j > 0)
        def _():
            st2[1:2, :] = st2[1:2, :] + t1

    @pl.when(p == 2)
    def _phase2():
        cnt = float(_N)
        mu2 = st2[0:1, :] / cnt
        dm2 = mu2 - st2[2:3, :]
        var2 = st2[1:2, :] / cnt - dm2 * dm2
        inv2 = lax.rsqrt(var2 + 1e-5)
        sn = (s_scr[pl.ds(row0, _AB), :] - mu2) * (g2_ref[:] * inv2) + be2_ref[:]
        xn = jax.nn.softplus(x_ref[pl.ds(row0, _AB), pl.ds(0, _D)] + sn)
        out_ref[:] = jnp.concatenate(
            [xn, jnp.zeros((_AB, _D), jnp.float32)], axis=1
        )


def _conv_layer(x, gathered, nbr2, w, b, g1, be1, g2, be2):
    ws = w[:_D]
    wn = jnp.concatenate(
        [w[_D : 2 * _D], jnp.zeros((_D, 2 * _D), jnp.float32)], axis=0
    )
    we = w[2 * _D :]
    eb = _AB * _M
    return pl.pallas_call(
        _conv_body,
        grid=(3, _NB),
        in_specs=[
            pl.BlockSpec((_N, 2 * _D), lambda p, j: (0, 0)),
            pl.BlockSpec((eb, 2 * _D), lambda p, j: (jnp.where(p == 2, 0, j), 0)),
            pl.BlockSpec((_AB, _M, _NBR),
                         lambda p, j: (jnp.where(p == 2, 0, j), 0, 0)),
            pl.BlockSpec((_D, 2 * _D), lambda p, j: (0, 0)),
            pl.BlockSpec((2 * _D, 2 * _D), lambda p, j: (0, 0)),
            pl.BlockSpec((_NBR, 2 * _D), lambda p, j: (0, 0)),
            pl.BlockSpec((1, 2 * _D), lambda p, j: (0, 0)),
            pl.BlockSpec((1, 2 * _D), lambda p, j: (0, 0)),
            pl.BlockSpec((1, 2 * _D), lambda p, j: (0, 0)),
            pl.BlockSpec((1, _D), lambda p, j: (0, 0)),
            pl.BlockSpec((1, _D), lambda p, j: (0, 0)),
        ],
        out_specs=pl.BlockSpec((_AB, 2 * _D), lambda p, j: (jnp.where(p == 2, j, 0), 0)),
        out_shape=jax.ShapeDtypeStruct((_N, 2 * _D), jnp.float32),
        scratch_shapes=[
            pltpu.VMEM((_N, _D), jnp.float32),
            pltpu.VMEM((8, 2 * _D), jnp.float32),
            pltpu.VMEM((8, _D), jnp.float32),
        ],
    )(x, gathered, nbr2, ws, wn, we, b.reshape(1, 2 * _D),
      g1.reshape(1, 2 * _D), be1.reshape(1, 2 * _D),
      g2.reshape(1, _D), be2.reshape(1, _D))


def _head_body(pa_ref, pb_ref, daw_ref, dab_ref, dbw_ref, dbb_ref,
               ffw_ref, ffb_ref, ow_ref, ob_ref, out_ref):
    ma = jnp.mean(pa_ref[:], axis=1)
    mb = jnp.mean(pb_ref[:], axis=1)
    ca = jax.nn.softplus(
        jnp.dot(ma, daw_ref[:], preferred_element_type=jnp.float32) + dab_ref[:]
    )
    cb = jax.nn.softplus(
        jnp.dot(mb, dbw_ref[:], preferred_element_type=jnp.float32) + dbb_ref[:]
    )
    crys = jnp.abs(ca - cb)
    h = jax.nn.softplus(
        jnp.dot(crys, ffw_ref[:], preferred_element_type=jnp.float32)
        + ffb_ref[:]
    )
    out_ref[:] = (
        jnp.dot(h, ow_ref[:], preferred_element_type=jnp.float32) + ob_ref[:]
    )


def _head(pa, pb, daw, dab, dbw, dbb, ffw, ffb, ow, ob, n0):
    return pl.pallas_call(
        _head_body,
        out_shape=jax.ShapeDtypeStruct((n0, 1), jnp.float32),
    )(pa, pb, daw, dab.reshape(1, _H), dbw, dbb.reshape(1, _H),
      ffw, ffb.reshape(1, _H), ow, ob.reshape(1, 1))


def _branch(atom_fea, nbr_fea, nbr_idx, cidx, emb_w, emb_b,
            cw, cb, cg1, cbe1, cg2, cbe2):
    x = _embed(atom_fea, emb_w, emb_b)
    idx_flat = nbr_idx.reshape(_N * _M).astype(jnp.int32)
    nconv = cw.shape[0]
    for i in range(nconv):
        gathered = _sc_gather(x, idx_flat)
        x = _conv_layer(x, gathered, nbr_fea, cw[i], cb[i],
                        cg1[i], cbe1[i], cg2[i], cbe2[i])
    n0, p_ = cidx.shape
    pool_n = n0 * p_
    pad = (-pool_n) % (_CHUNK * 2)
    cflat = cidx.reshape(pool_n).astype(jnp.int32)
    if pad:
        cflat = jnp.concatenate([cflat, jnp.zeros((pad,), jnp.int32)])
    pooled = _sc_gather(x, cflat)[:pool_n, :_D]
    return pooled.reshape(n0, p_, _D)


def kernel(atom_fea_A, nbr_fea_A, nbr_fea_idx_A, crystal_atom_idx_A,
           atom_fea_B, nbr_fea_B, nbr_fea_idx_B, crystal_atom_idx_B,
           emb_A_W, emb_A_b, emb_B_W, emb_B_b,
           conv_A_W, conv_A_b, conv_A_g1, conv_A_be1, conv_A_g2, conv_A_be2,
           conv_B_W, conv_B_b, conv_B_g1, conv_B_be1, conv_B_g2, conv_B_be2,
           dense_A_W, dense_A_b, dense_B_W, dense_B_b,
           ff_W, ff_b, out_W, out_b):
    pa = _branch(atom_fea_A, nbr_fea_A, nbr_fea_idx_A, crystal_atom_idx_A,
                 emb_A_W, emb_A_b, conv_A_W, conv_A_b,
                 conv_A_g1, conv_A_be1, conv_A_g2, conv_A_be2)
    pb = _branch(atom_fea_B, nbr_fea_B, nbr_fea_idx_B, crystal_atom_idx_B,
                 emb_B_W, emb_B_b, conv_B_W, conv_B_b,
                 conv_B_g1, conv_B_be1, conv_B_g2, conv_B_be2)
    n0 = crystal_atom_idx_A.shape[0]
    return _head(pa, pb, dense_A_W, dense_A_b, dense_B_W, dense_B_b,
                 ff_W[0], ff_b[0], out_W, out_b, n0)


# speedup vs baseline: 1.1583x; 1.1583x over previous
import functools

import jax
import jax.numpy as jnp
from jax import lax
from jax.experimental import pallas as pl
from jax.experimental.pallas import tpu as pltpu
from jax.experimental.pallas import tpu_sc as plsc

_N = 10000
_M = 16
_D = 64
_NBR = 41
_H = 128
_NW = 32
_CHUNK = 128
_AB = 400
_NB = _N // _AB


_G = 3


def _sc_gather(table, idx_flat):
    e = idx_flat.shape[0]
    v, dpad = table.shape
    assert e % _CHUNK == 0 and dpad == 2 * _D
    c_total = e // _CHUNK
    idx2 = idx_flat.reshape(c_total, _CHUNK)
    c_base, c_extra = divmod(c_total, _NW)

    mesh = plsc.VectorSubcoreMesh(core_axis_name="c", subcore_axis_name="s")

    @functools.partial(
        pl.kernel,
        mesh=mesh,
        out_type=jax.ShapeDtypeStruct((e, 2 * _D), jnp.float32),
        scratch_types=[
            pltpu.VMEM((_CHUNK,), jnp.int32),
            pltpu.VMEM((_CHUNK, 2 * _D), jnp.float32),
            pltpu.SemaphoreType.DMA,
        ],
    )
    def k(table_hbm, idx_hbm, out_hbm, idx_v, rows_v, sem):
        wid = lax.axis_index("s") * 2 + lax.axis_index("c")
        n_w = c_base + jnp.where(wid < c_extra, 1, 0)

        def body(i, carry):
            cid = wid + i * _NW
            pltpu.sync_copy(idx_hbm.at[cid], idx_v)
            pltpu.async_copy(table_hbm.at[idx_v], rows_v, sem).wait()
            pltpu.sync_copy(rows_v, out_hbm.at[pl.ds(cid * _CHUNK, _CHUNK)])
            return carry

        lax.fori_loop(0, n_w, body, 0)

    return k(table, idx2)


def _emb_body(af_ref, w_ref, b_ref, out_ref):
    x = (
        jnp.dot(af_ref[:], w_ref[:], preferred_element_type=jnp.float32)
        + b_ref[:]
    )
    out_ref[:] = jnp.concatenate(
        [x, jnp.zeros((x.shape[0], _D), jnp.float32)], axis=1
    )


def _embed(atom_fea, w, b):
    return pl.pallas_call(
        _emb_body,
        out_shape=jax.ShapeDtypeStruct((_N, 2 * _D), jnp.float32),
    )(atom_fea, w, b.reshape(1, _D))


def _conv_body(x_ref, g_ref, nbr_ref, ws_ref, wn_ref, we_ref, b_ref,
               g1_ref, be1_ref, g2_ref, be2_ref, out_ref, s_scr, st1, st2):
    p = pl.program_id(0)
    j = pl.program_id(1)
    row0 = pl.multiple_of(j * _AB, 8)

    def edge_parts():
        xs = x_ref[pl.ds(row0, _AB), pl.ds(0, _D)]
        zself = (
            jnp.dot(xs, ws_ref[:], preferred_element_type=jnp.float32)
            + b_ref[:]
        )
        ze = jnp.dot(g_ref[:], wn_ref[:], preferred_element_type=jnp.float32)
        nbr2 = nbr_ref[:].reshape(_AB * _M, _NBR)
        ze = ze + jnp.dot(
            nbr2, we_ref[:], preferred_element_type=jnp.float32
        )
        return zself, ze

    @pl.when(p == 0)
    def _phase0():
        zself, ze = edge_parts()
        sze = jnp.sum(ze.reshape(_AB, _M, 2 * _D), axis=1)
        s0 = (
            jnp.sum(ze, axis=0, keepdims=True)
            + _M * jnp.sum(zself, axis=0, keepdims=True)
        )

        @pl.when(j == 0)
        def _():
            st1[0:1, :] = s0
            st1[2:3, :] = s0 / float(_AB * _M)

        @pl.when(j > 0)
        def _():
            st1[0:1, :] = st1[0:1, :] + s0

        zc = zself - st1[2:3, :]
        s1 = (
            jnp.sum(ze * ze, axis=0, keepdims=True)
            + 2.0 * jnp.sum(zc * sze, axis=0, keepdims=True)
            + _M * jnp.sum(zc * zc, axis=0, keepdims=True)
        )

        @pl.when(j == 0)
        def _():
            st1[1:2, :] = s1

        @pl.when(j > 0)
        def _():
            st1[1:2, :] = st1[1:2, :] + s1

    @pl.when(p == 1)
    def _phase1():
        cnt = float(_N * _M)
        mu = st1[0:1, :] / cnt
        dm = mu - st1[2:3, :]
        var = st1[1:2, :] / cnt - dm * dm
        inv = lax.rsqrt(var + 1e-5)
        scale = g1_ref[:] * inv

        zself, ze = edge_parts()
        zc3 = ze.reshape(_AB, _M, 2 * _D) + (zself - mu)[:, None, :]
        zn = zc3 * scale[None] + be1_ref[:][None]
        filt = jax.nn.sigmoid(zn[:, :, :_D])
        core = jax.nn.softplus(zn[:, :, _D:])
        sblk = jnp.sum(filt * core, axis=1)
        s_scr[pl.ds(row0, _AB), :] = sblk
        t0 = jnp.sum(sblk, axis=0, keepdims=True)

        @pl.when(j == 0)
        def _():
            st2[0:1, :] = t0
            st2[2:3, :] = t0 / float(_AB)

        @pl.when(j > 0)
        def _():
            st2[0:1, :] = st2[0:1, :] + t0

        sc = sblk - st2[2:3, :]
        t1 = jnp.sum(sc * sc, axis=0, keepdims=True)

        @pl.when(j == 0)
        def _():
            st2[1:2, :] = t1

        @pl.when(j > 0)
        def _():
            st2[1:2, :] = st2[1:2, :] + t1

    @pl.when(p == 2)
    def _phase2():
        cnt = float(_N)
        mu2 = st2[0:1, :] / cnt
        dm2 = mu2 - st2[2:3, :]
        var2 = st2[1:2, :] / cnt - dm2 * dm2
        inv2 = lax.rsqrt(var2 + 1e-5)
        sn = (s_scr[pl.ds(row0, _AB), :] - mu2) * (g2_ref[:] * inv2) + be2_ref[:]
        xn = jax.nn.softplus(x_ref[pl.ds(row0, _AB), pl.ds(0, _D)] + sn)
        out_ref[:] = jnp.concatenate(
            [xn, jnp.zeros((_AB, _D), jnp.float32)], axis=1
        )


def _conv_layer(x, gathered, nbr2, w, b, g1, be1, g2, be2):
    ws = w[:_D]
    wn = jnp.concatenate(
        [w[_D : 2 * _D], jnp.zeros((_D, 2 * _D), jnp.float32)], axis=0
    )
    we = w[2 * _D :]
    eb = _AB * _M
    return pl.pallas_call(
        _conv_body,
        grid=(3, _NB),
        in_specs=[
            pl.BlockSpec((_N, 2 * _D), lambda p, j: (0, 0)),
            pl.BlockSpec((eb, 2 * _D), lambda p, j: (jnp.where(p == 2, 0, j), 0)),
            pl.BlockSpec((_AB, _M, _NBR),
                         lambda p, j: (jnp.where(p == 2, 0, j), 0, 0)),
            pl.BlockSpec((_D, 2 * _D), lambda p, j: (0, 0)),
            pl.BlockSpec((2 * _D, 2 * _D), lambda p, j: (0, 0)),
            pl.BlockSpec((_NBR, 2 * _D), lambda p, j: (0, 0)),
            pl.BlockSpec((1, 2 * _D), lambda p, j: (0, 0)),
            pl.BlockSpec((1, 2 * _D), lambda p, j: (0, 0)),
            pl.BlockSpec((1, 2 * _D), lambda p, j: (0, 0)),
            pl.BlockSpec((1, _D), lambda p, j: (0, 0)),
            pl.BlockSpec((1, _D), lambda p, j: (0, 0)),
        ],
        out_specs=pl.BlockSpec((_AB, 2 * _D), lambda p, j: (jnp.where(p == 2, j, 0), 0)),
        out_shape=jax.ShapeDtypeStruct((_N, 2 * _D), jnp.float32),
        scratch_shapes=[
            pltpu.VMEM((_N, _D), jnp.float32),
            pltpu.VMEM((8, 2 * _D), jnp.float32),
            pltpu.VMEM((8, _D), jnp.float32),
        ],
    )(x, gathered, nbr2, ws, wn, we, b.reshape(1, 2 * _D),
      g1.reshape(1, 2 * _D), be1.reshape(1, 2 * _D),
      g2.reshape(1, _D), be2.reshape(1, _D))


def _head_body(pa_ref, pb_ref, daw_ref, dab_ref, dbw_ref, dbb_ref,
               ffw_ref, ffb_ref, ow_ref, ob_ref, out_ref):
    ma = jnp.mean(pa_ref[:], axis=1)
    mb = jnp.mean(pb_ref[:], axis=1)
    ca = jax.nn.softplus(
        jnp.dot(ma, daw_ref[:], preferred_element_type=jnp.float32) + dab_ref[:]
    )
    cb = jax.nn.softplus(
        jnp.dot(mb, dbw_ref[:], preferred_element_type=jnp.float32) + dbb_ref[:]
    )
    crys = jnp.abs(ca - cb)
    h = jax.nn.softplus(
        jnp.dot(crys, ffw_ref[:], preferred_element_type=jnp.float32)
        + ffb_ref[:]
    )
    out_ref[:] = (
        jnp.dot(h, ow_ref[:], preferred_element_type=jnp.float32) + ob_ref[:]
    )


def _head(pa, pb, daw, dab, dbw, dbb, ffw, ffb, ow, ob, n0):
    return pl.pallas_call(
        _head_body,
        out_shape=jax.ShapeDtypeStruct((n0, 1), jnp.float32),
    )(pa, pb, daw, dab.reshape(1, _H), dbw, dbb.reshape(1, _H),
      ffw, ffb.reshape(1, _H), ow, ob.reshape(1, 1))


def _branch(atom_fea, nbr_fea, nbr_idx, cidx, emb_w, emb_b,
            cw, cb, cg1, cbe1, cg2, cbe2):
    x = _embed(atom_fea, emb_w, emb_b)
    idx_flat = nbr_idx.reshape(_N * _M).astype(jnp.int32)
    nconv = cw.shape[0]
    for i in range(nconv):
        gathered = _sc_gather(x, idx_flat)
        x = _conv_layer(x, gathered, nbr_fea, cw[i], cb[i],
                        cg1[i], cbe1[i], cg2[i], cbe2[i])
    n0, p_ = cidx.shape
    pool_n = n0 * p_
    pad = (-pool_n) % (_CHUNK * 2)
    cflat = cidx.reshape(pool_n).astype(jnp.int32)
    if pad:
        cflat = jnp.concatenate([cflat, jnp.zeros((pad,), jnp.int32)])
    pooled = _sc_gather(x, cflat)[:pool_n, :_D]
    return pooled.reshape(n0, p_, _D)


def kernel(atom_fea_A, nbr_fea_A, nbr_fea_idx_A, crystal_atom_idx_A,
           atom_fea_B, nbr_fea_B, nbr_fea_idx_B, crystal_atom_idx_B,
           emb_A_W, emb_A_b, emb_B_W, emb_B_b,
           conv_A_W, conv_A_b, conv_A_g1, conv_A_be1, conv_A_g2, conv_A_be2,
           conv_B_W, conv_B_b, conv_B_g1, conv_B_be1, conv_B_g2, conv_B_be2,
           dense_A_W, dense_A_b, dense_B_W, dense_B_b,
           ff_W, ff_b, out_W, out_b):
    pa = _branch(atom_fea_A, nbr_fea_A, nbr_fea_idx_A, crystal_atom_idx_A,
                 emb_A_W, emb_A_b, conv_A_W, conv_A_b,
                 conv_A_g1, conv_A_be1, conv_A_g2, conv_A_be2)
    pb = _branch(atom_fea_B, nbr_fea_B, nbr_fea_idx_B, crystal_atom_idx_B,
                 emb_B_W, emb_B_b, conv_B_W, conv_B_b,
                 conv_B_g1, conv_B_be1, conv_B_g2, conv_B_be2)
    n0 = crystal_atom_idx_A.shape[0]
    return _head(pa, pb, dense_A_W, dense_A_b, dense_B_W, dense_B_b,
                 ff_W[0], ff_b[0], out_W, out_b, n0)
